# baseline (speedup 1.0000x reference)
import functools

import jax
import jax.numpy as jnp
from jax.experimental import pallas as pl

F = 51
FP = 64
K = 128
BINS = 2
T = 2048


def _fused_kernel(prev_ref, cur_ref, next_ref,
                  w0n_ref, w0p_ref, w1n_ref, w1p_ref, w1y_ref,
                  whn_ref, whp_ref, why_ref,
                  b0_ref, b1_ref, bh_ref,
                  out_ref, *, n_rows):
    i = pl.program_id(0)
    f32 = jnp.float32
    xh = jnp.concatenate(
        [prev_ref[T - 3:, :], cur_ref[...], next_ref[:3, :]], axis=0)
    xneg = jnp.minimum(xh, 0.0)
    xpos = jnp.maximum(xh, 0.0)

    acc0 = jnp.broadcast_to(b0_ref[...], (T + 4, K)).astype(f32)
    for t in range(3):
        acc0 = acc0 + jnp.dot(xneg[t:t + T + 4], w0n_ref[t],
                              preferred_element_type=f32)
        acc0 = acc0 + jnp.dot(xpos[t:t + T + 4], w0p_ref[t],
                              preferred_element_type=f32)
    y0 = jnp.maximum(acc0, 0.0)
    gr = i * T - 2 + jax.lax.broadcasted_iota(jnp.int32, (T + 4, 1), 0)
    y0 = jnp.where((gr >= 0) & (gr < n_rows), y0, 0.0)
    y0 = y0.astype(xh.dtype)

    acc1 = jnp.broadcast_to(b1_ref[...], (T, K)).astype(f32)
    for t in range(5):
        acc1 = acc1 + jnp.dot(xneg[1 + t:1 + t + T], w1n_ref[t],
                              preferred_element_type=f32)
        acc1 = acc1 + jnp.dot(xpos[1 + t:1 + t + T], w1p_ref[t],
                              preferred_element_type=f32)
        acc1 = acc1 + jnp.dot(y0[t:t + T], w1y_ref[t],
                              preferred_element_type=f32)
    y1 = jnp.maximum(acc1, 0.0).astype(xh.dtype)

    z = (jnp.dot(xneg[3:3 + T], whn_ref[...], preferred_element_type=f32)
         + jnp.dot(xpos[3:3 + T], whp_ref[...], preferred_element_type=f32)
         + jnp.dot(y1, why_ref[...], preferred_element_type=f32)
         + bh_ref[...])
    out_ref[...] = jax.nn.softplus(z)


def _pad_cin(w, cin_pad):
    pad = [(0, 0)] * (w.ndim - 2) + [(0, cin_pad - w.shape[-2]), (0, 0)]
    return jnp.pad(w, pad)


@functools.partial(jax.jit, static_argnums=())
def kernel(inputs, W0, b0, W1, b1, Wh, bh):
    x = inputs[0]
    n, f = x.shape
    nb = n // T
    cdt = jnp.bfloat16
    xpad = jnp.zeros((n + 2 * T, FP), cdt).at[T:T + n, :f].set(x.astype(cdt))

    w0n = _pad_cin(W0[:, :F, :], FP).astype(cdt)
    w0p = _pad_cin(W0[:, F:2 * F, :], FP).astype(cdt)
    w1n = _pad_cin(W1[:, :F, :], FP).astype(cdt)
    w1p = _pad_cin(W1[:, F:2 * F, :], FP).astype(cdt)
    w1y = W1[:, 2 * F:, :].astype(cdt)
    whn = _pad_cin(Wh[:F, :], FP).astype(cdt)
    whp = _pad_cin(Wh[F:2 * F, :], FP).astype(cdt)
    why = Wh[2 * F:, :].astype(cdt)
    b0r = b0.reshape(1, K)
    b1r = b1.reshape(1, K)
    bhr = bh.reshape(1, BINS)

    full = lambda shape: pl.BlockSpec(shape, lambda i: (0,) * len(shape))
    out = pl.pallas_call(
        functools.partial(_fused_kernel, n_rows=n),
        grid=(nb,),
        in_specs=[
            pl.BlockSpec((T, FP), lambda i: (i, 0)),
            pl.BlockSpec((T, FP), lambda i: (i + 1, 0)),
            pl.BlockSpec((T, FP), lambda i: (i + 2, 0)),
            full((3, FP, K)), full((3, FP, K)),
            full((5, FP, K)), full((5, FP, K)), full((5, K, K)),
            full((FP, BINS)), full((FP, BINS)), full((K, BINS)),
            full((1, K)), full((1, K)), full((1, BINS)),
        ],
        out_specs=pl.BlockSpec((T, BINS), lambda i: (i, 0)),
        out_shape=jax.ShapeDtypeStruct((n, BINS), jnp.float32),
    )(xpad, xpad, xpad, w0n, w0p, w1n, w1p, w1y, whn, whp, why,
      b0r, b1r, bhr)
    return out[None, :, :]


# speedup vs baseline: 1.3164x; 1.3164x over previous
import functools

import jax
import jax.numpy as jnp
from jax.experimental import pallas as pl

F = 51
FP = 64
K = 128
BINS = 2
G = 2
T = 2048
R = T // G


def _fused_kernel(prev_ref, cur_ref, next_ref,
                  w0b_ref, w1xb_ref, w1yb_ref, whx_ref, why_ref,
                  b0g_ref, b1g_ref, bhg_ref,
                  out_ref, *, n_groups):
    i = pl.program_id(0)
    f32 = jnp.float32
    xe = jnp.concatenate(
        [prev_ref[R - 2:, :], cur_ref[...], next_ref[:2, :]], axis=0)
    xc = jnp.concatenate(
        [jnp.minimum(xe, 0.0), jnp.maximum(xe, 0.0)], axis=1)

    acc0 = jnp.broadcast_to(b0g_ref[...], (R + 2, G * K)).astype(f32)
    for o in range(3):
        acc0 = acc0 + jnp.dot(xc[o:o + R + 2], w0b_ref[o],
                              preferred_element_type=f32)
    y0 = jnp.maximum(acc0, 0.0)
    ge = i * R - 1 + jax.lax.broadcasted_iota(jnp.int32, (R + 2, 1), 0)
    y0 = jnp.where((ge >= 0) & (ge < n_groups), y0, 0.0)
    y0 = y0.astype(xe.dtype)

    acc1 = jnp.broadcast_to(b1g_ref[...], (R, G * K)).astype(f32)
    for o in range(3):
        acc1 = acc1 + jnp.dot(xc[1 + o:1 + o + R], w1xb_ref[o],
                              preferred_element_type=f32)
        acc1 = acc1 + jnp.dot(y0[o:o + R], w1yb_ref[o],
                              preferred_element_type=f32)
    y1 = jnp.maximum(acc1, 0.0).astype(xe.dtype)

    z = (jnp.dot(xc[2:2 + R], whx_ref[...], preferred_element_type=f32)
         + jnp.dot(y1, why_ref[...], preferred_element_type=f32)
         + bhg_ref[...])
    out_ref[...] = jax.nn.softplus(z)


def kernel(inputs, W0, b0, W1, b1, Wh, bh):
    x = inputs[0]
    n, f = x.shape
    nb = n // T
    ng = n // G
    cdt = jnp.bfloat16

    xp = jnp.zeros((n, FP), cdt).at[:, :f].set(x.astype(cdt))
    xg = xp.reshape(ng, G * FP)
    xgpad = jnp.zeros((ng + 2 * R, G * FP), cdt).at[R:R + ng, :].set(xg)

    w0b = jnp.zeros((3, 2 * G * FP, G * K), jnp.float32)
    w1xb = jnp.zeros((3, 2 * G * FP, G * K), jnp.float32)
    w1yb = jnp.zeros((3, G * K, G * K), jnp.float32)
    for o in (-1, 0, 1):
        for s in range(G):
            for r in range(G):
                t0 = G * o + r - s + 1
                if 0 <= t0 < 3:
                    for part in range(2):
                        w0b = w0b.at[
                            o + 1,
                            part * G * FP + r * FP:part * G * FP + r * FP + f,
                            s * K:(s + 1) * K,
                        ].set(W0[t0, part * f:(part + 1) * f, :])
                t1 = G * o + r - s + 2
                if 0 <= t1 < 5:
                    for part in range(2):
                        w1xb = w1xb.at[
                            o + 1,
                            part * G * FP + r * FP:part * G * FP + r * FP + f,
                            s * K:(s + 1) * K,
                        ].set(W1[t1, part * f:(part + 1) * f, :])
                    w1yb = w1yb.at[
                        o + 1, r * K:(r + 1) * K, s * K:(s + 1) * K,
                    ].set(W1[t1, 2 * f:, :])
    whx = jnp.zeros((2 * G * FP, G * BINS), jnp.float32)
    why = jnp.zeros((G * K, G * BINS), jnp.float32)
    for r in range(G):
        for part in range(2):
            whx = whx.at[
                part * G * FP + r * FP:part * G * FP + r * FP + f,
                r * BINS:(r + 1) * BINS,
            ].set(Wh[part * f:(part + 1) * f, :])
        why = why.at[r * K:(r + 1) * K, r * BINS:(r + 1) * BINS].set(
            Wh[2 * f:, :])

    w0b, w1xb, w1yb = w0b.astype(cdt), w1xb.astype(cdt), w1yb.astype(cdt)
    whx, why = whx.astype(cdt), why.astype(cdt)
    b0g = jnp.tile(b0, G).reshape(1, G * K)
    b1g = jnp.tile(b1, G).reshape(1, G * K)
    bhg = jnp.tile(bh, G).reshape(1, G * BINS)

    full = lambda shape: pl.BlockSpec(shape, lambda i: (0,) * len(shape))
    out = pl.pallas_call(
        functools.partial(_fused_kernel, n_groups=ng),
        grid=(nb,),
        in_specs=[
            pl.BlockSpec((R, G * FP), lambda i: (i, 0)),
            pl.BlockSpec((R, G * FP), lambda i: (i + 1, 0)),
            pl.BlockSpec((R, G * FP), lambda i: (i + 2, 0)),
            full((3, 2 * G * FP, G * K)), full((3, 2 * G * FP, G * K)),
            full((3, G * K, G * K)),
            full((2 * G * FP, G * BINS)), full((G * K, G * BINS)),
            full((1, G * K)), full((1, G * K)), full((1, G * BINS)),
        ],
        out_specs=pl.BlockSpec((R, G * BINS), lambda i: (i, 0)),
        out_shape=jax.ShapeDtypeStruct((ng, G * BINS), jnp.float32),
    )(xgpad, xgpad, xgpad, w0b, w1xb, w1yb, whx, why, b0g, b1g, bhg)
    return out.reshape(n, BINS)[None]
